# baseline (speedup 1.0000x reference)
import functools

import jax
import jax.numpy as jnp
from jax import lax
from jax.experimental import pallas as pl
from jax.experimental.pallas import tpu as pltpu
from jax.experimental.pallas import tpu_sc as plsc

NC = 2
NS = 16
NW = NC * NS

BATCH = 4096
MAX_LEN = 200
DIM = 64
B = BATCH * MAX_LEN
BPW = B // NW
G = 512
NB = 2
NCH = BPW // G
NT = (NCH - 2 * NB) // NB
N_EMBS = 1000002

assert NCH == NB * (NT + 2)


def _body(ids_hbm, table_hbm, out_hbm, idx_v, *rest):
    rows = rest[:NB]
    gsem = rest[NB:2 * NB]
    osem = rest[2 * NB:3 * NB]

    wid = lax.axis_index("s") * NC + lax.axis_index("c")
    base = wid * BPW
    pltpu.sync_copy(ids_hbm.at[pl.ds(base, BPW)], idx_v)

    def fire(c, b):
        pltpu.async_copy(
            table_hbm.at[pl.ds(base + c * G, G)], rows[b], gsem[b])

    def wait_g(c, b):
        pltpu.make_async_copy(
            table_hbm.at[pl.ds(base + c * G, G)], rows[b], gsem[b]).wait()

    def wout(c, b):
        pltpu.async_copy(
            rows[b], out_hbm.at[idx_v.at[pl.ds(c * G, G)]], osem[b])

    def wait_o(c, b):
        pltpu.make_async_copy(
            rows[b], out_hbm.at[idx_v.at[pl.ds(c * G, G)]], osem[b]).wait()

    for c in range(NB - 1):
        fire(c, c)
    for c in range(NB):
        wait_g(c, c)
        wout(c, c)
        if c >= 1:
            wait_o(c - 1, c - 1)
        fire(c + NB - 1, (c + NB - 1) % NB)

    def loop_body(t, carry):
        cb = NB * t + NB
        for k in range(NB):
            c = cb + k
            wait_g(c, k)
            wout(c, k)
            wait_o(c - 1, (k - 1) % NB)
            fire(c + NB - 1, (k - 1) % NB)
        return carry

    lax.fori_loop(0, NT, loop_body, 0)

    for c in range(NCH - NB, NCH):
        wait_g(c, c % NB)
        wout(c, c % NB)
        wait_o(c - 1, (c - 1) % NB)
        if c + NB - 1 < NCH:
            fire(c + NB - 1, (c + NB - 1) % NB)
    wait_o(NCH - 1, (NCH - 1) % NB)


@jax.jit
def _gather(ids_flat, table):
    mesh = plsc.VectorSubcoreMesh(
        core_axis_name="c", subcore_axis_name="s",
        num_cores=NC, num_subcores=NS)
    run = functools.partial(
        pl.kernel, mesh=mesh,
        compiler_params=pltpu.CompilerParams(use_tc_tiling_on_sc=False),
        out_type=jax.ShapeDtypeStruct((N_EMBS, DIM), jnp.float32),
        scratch_types=(
            [pltpu.VMEM((BPW,), jnp.int32)]
            + [pltpu.VMEM((G, DIM), jnp.float32) for _ in range(NB)]
            + [pltpu.SemaphoreType.DMA for _ in range(2 * NB)]
        ))(_body)
    return run(ids_flat, table)


def kernel(torch_ids, pads, table):
    ids_flat = torch_ids.reshape(-1)
    out = _gather(ids_flat, table)
    return out[:16, :].reshape(16, 1, 64), pads


# speedup vs baseline: 2.9823x; 2.9823x over previous
import functools

import jax
import jax.numpy as jnp
from jax import lax
from jax.experimental import pallas as pl
from jax.experimental.pallas import tpu as pltpu
from jax.experimental.pallas import tpu_sc as plsc

NC = 2
NS = 16
NW = NC * NS

BATCH = 4096
MAX_LEN = 200
DIM = 64
B = BATCH * MAX_LEN
RPW = BATCH // NW
BPW = B // NW
S = 40
NCK = MAX_LEN // S
PADDING_ID = 1000001


def _body(ids_hbm, pads_hbm, table_hbm, out_hbm,
          idx_v, pads_v, padidx_v, padfill_v, rows_v,
          gsem, osem, psem):
    wid = lax.axis_index("s") * NC + lax.axis_index("c")
    base = wid * BPW
    pltpu.sync_copy(ids_hbm.at[pl.ds(base, BPW)], idx_v)
    pltpu.sync_copy(pads_hbm.at[pl.ds(wid * RPW, RPW)], pads_v)

    pid = jnp.full((16,), PADDING_ID, dtype=jnp.int32)
    for off in range(0, 48, 16):
        padidx_v[pl.ds(off, 16)] = pid
    pltpu.async_copy(
        table_hbm.at[padidx_v.at[pl.ds(0, S)]], padfill_v, psem).wait()

    lanes = lax.iota(jnp.int32, 16)

    def trip_count(r):
        pvec = pads_v[pl.ds(16 * (r // 16), 16)]
        tvec = (MAX_LEN + S - 1 - pvec) // S
        return jnp.max(jnp.where(lanes == (r % 16), tvec, 0))

    def drain_write(vo):
        pltpu.make_async_copy(
            rows_v.at[pl.ds(vo, S)], out_hbm.at[pl.ds(base, S)], osem).wait()

    def do_row(r):
        t = trip_count(r)
        o = base + r * MAX_LEN
        vo = (r % 2) * MAX_LEN

        def fire(k, carry):
            pltpu.async_copy(
                table_hbm.at[idx_v.at[pl.ds(r * MAX_LEN + k * S, S)]],
                rows_v.at[pl.ds(vo + k * S, S)], gsem)
            return carry

        lax.fori_loop(0, t, fire, 0)

        def wait_and_write(k, carry):
            pltpu.make_async_copy(
                table_hbm.at[idx_v.at[pl.ds(r * MAX_LEN + k * S, S)]],
                rows_v.at[pl.ds(vo + k * S, S)], gsem).wait()
            pltpu.async_copy(
                rows_v.at[pl.ds(vo + k * S, S)],
                out_hbm.at[pl.ds(o + k * S, S)], osem)
            return carry

        lax.fori_loop(0, t, wait_and_write, 0)

        def fill(k, carry):
            pltpu.async_copy(
                padfill_v, out_hbm.at[pl.ds(o + k * S, S)], osem)
            return carry

        lax.fori_loop(t, NCK, fill, 0)

    do_row(0)
    do_row(1)

    def row_loop(r, carry):
        for _ in range(NCK):
            drain_write((r % 2) * MAX_LEN)
        do_row(r)
        return carry

    lax.fori_loop(2, RPW, row_loop, 0)

    for r in (RPW - 2, RPW - 1):
        for _ in range(NCK):
            drain_write((r % 2) * MAX_LEN)


@jax.jit
def _gather(ids_flat, pads, table):
    mesh = plsc.VectorSubcoreMesh(
        core_axis_name="c", subcore_axis_name="s",
        num_cores=NC, num_subcores=NS)
    run = functools.partial(
        pl.kernel, mesh=mesh,
        compiler_params=pltpu.CompilerParams(use_tc_tiling_on_sc=False, needs_layout_passes=False),
        out_type=jax.ShapeDtypeStruct((B, DIM), jnp.float32),
        scratch_types=(
            [pltpu.VMEM((BPW,), jnp.int32),
             pltpu.VMEM((RPW,), jnp.int32),
             pltpu.VMEM((48,), jnp.int32),
             pltpu.VMEM((S, DIM), jnp.float32),
             pltpu.VMEM((2 * MAX_LEN, DIM), jnp.float32),
             pltpu.SemaphoreType.DMA,
             pltpu.SemaphoreType.DMA,
             pltpu.SemaphoreType.DMA]
        ))(_body)
    return run(ids_flat, pads, table)


def kernel(torch_ids, pads, table):
    ids_flat = torch_ids.reshape(-1)
    out = _gather(ids_flat, pads, table)
    return out.reshape(BATCH, MAX_LEN, DIM), pads


# speedup vs baseline: 5.4169x; 1.8164x over previous
import functools

import jax
import jax.numpy as jnp
from jax import lax
from jax.experimental import pallas as pl
from jax.experimental.pallas import tpu as pltpu
from jax.experimental.pallas import tpu_sc as plsc

NC = 2
NS = 16
NW = NC * NS

BATCH = 4096
MAX_LEN = 200
DIM = 64
B = BATCH * MAX_LEN
RPW = BATCH // NW
BPW = B // NW
S = 8
NCK = MAX_LEN // S
PADDING_ID = 1000001


def _body(ids_hbm, pads_hbm, table_hbm, out_hbm,
          idx_v, pads_v, padidx_v, padfill_v, rows_v,
          gsem, osem, psem):
    wid = lax.axis_index("s") * NC + lax.axis_index("c")
    base = wid * BPW
    pltpu.sync_copy(ids_hbm.at[pl.ds(base, BPW)], idx_v)
    pltpu.sync_copy(pads_hbm.at[pl.ds(wid * RPW, RPW)], pads_v)

    pid = jnp.full((16,), PADDING_ID, dtype=jnp.int32)
    for off in range(0, 48, 16):
        padidx_v[pl.ds(off, 16)] = pid
    pltpu.async_copy(
        table_hbm.at[padidx_v.at[pl.ds(0, S)]], padfill_v, psem).wait()

    lanes = lax.iota(jnp.int32, 16)

    def trip_count(r):
        pvec = pads_v[pl.ds(16 * (r // 16), 16)]
        tvec = (MAX_LEN + S - 1 - pvec) // S
        return jnp.max(jnp.where(lanes == (r % 16), tvec, 0))

    def drain_write(vo):
        pltpu.make_async_copy(
            rows_v.at[pl.ds(vo, S)], out_hbm.at[pl.ds(base, S)], osem).wait()

    def do_row(r):
        t = trip_count(r)
        o = base + r * MAX_LEN
        vo = (r % 2) * MAX_LEN

        def fire(k, carry):
            pltpu.async_copy(
                table_hbm.at[idx_v.at[pl.ds(r * MAX_LEN + k * S, S)]],
                rows_v.at[pl.ds(vo + k * S, S)], gsem)
            return carry

        lax.fori_loop(0, t, fire, 0)

        def wait_and_write(k, carry):
            pltpu.make_async_copy(
                table_hbm.at[idx_v.at[pl.ds(r * MAX_LEN + k * S, S)]],
                rows_v.at[pl.ds(vo + k * S, S)], gsem).wait()
            pltpu.async_copy(
                rows_v.at[pl.ds(vo + k * S, S)],
                out_hbm.at[pl.ds(o + k * S, S)], osem)
            return carry

        lax.fori_loop(0, t, wait_and_write, 0)

        def fill(k, carry):
            pltpu.async_copy(
                padfill_v, out_hbm.at[pl.ds(o + k * S, S)], osem)
            return carry

        lax.fori_loop(t, NCK, fill, 0)

    do_row(0)
    do_row(1)

    def row_loop(r, carry):
        for _ in range(NCK):
            drain_write((r % 2) * MAX_LEN)
        do_row(r)
        return carry

    lax.fori_loop(2, RPW, row_loop, 0)

    for r in (RPW - 2, RPW - 1):
        for _ in range(NCK):
            drain_write((r % 2) * MAX_LEN)


@jax.jit
def _gather(ids_flat, pads, table):
    mesh = plsc.VectorSubcoreMesh(
        core_axis_name="c", subcore_axis_name="s",
        num_cores=NC, num_subcores=NS)
    run = functools.partial(
        pl.kernel, mesh=mesh,
        compiler_params=pltpu.CompilerParams(use_tc_tiling_on_sc=False, needs_layout_passes=False),
        out_type=jax.ShapeDtypeStruct((B, DIM), jnp.float32),
        scratch_types=(
            [pltpu.VMEM((BPW,), jnp.int32),
             pltpu.VMEM((RPW,), jnp.int32),
             pltpu.VMEM((48,), jnp.int32),
             pltpu.VMEM((S, DIM), jnp.float32),
             pltpu.VMEM((2 * MAX_LEN, DIM), jnp.float32),
             pltpu.SemaphoreType.DMA,
             pltpu.SemaphoreType.DMA,
             pltpu.SemaphoreType.DMA]
        ))(_body)
    return run(ids_flat, pads, table)


def kernel(torch_ids, pads, table):
    ids_flat = torch_ids.reshape(-1)
    out = _gather(ids_flat, pads, table)
    return out.reshape(BATCH, MAX_LEN, DIM), pads
